# baseline (speedup 1.0000x reference)
import functools

import jax
import jax.numpy as jnp
from jax.experimental import pallas as pl
from jax.experimental.pallas import tpu as pltpu
from jax.experimental.pallas import tpu_sc as plsc

_GATHER_WINDOW = 128
_GATHER_K = 8


def _sc_gather(table, ids):
    num = ids.shape[0]
    depth = table.shape[1]
    step_rows = _GATHER_WINDOW * _GATHER_K
    ids2 = ids.reshape(1, num)
    mesh = plsc.VectorSubcoreMesh(core_axis_name="c", subcore_axis_name="s")

    @functools.partial(
        pl.kernel,
        out_type=jax.ShapeDtypeStruct((num, depth), table.dtype),
        mesh=mesh,
        scratch_types=[pltpu.SemaphoreType.DMA],
        compiler_params=pltpu.CompilerParams(use_tc_tiling_on_sc=False),
    )
    def gather_kernel(table_hbm, ids_hbm, out_hbm, sem):
        def body(i_vmem, o_vmem):
            copies = []
            for j in range(_GATHER_K):
                sl = pl.ds(j * _GATHER_WINDOW, _GATHER_WINDOW)
                copies.append(
                    pltpu.make_async_copy(
                        table_hbm.at[i_vmem.at[0, sl]], o_vmem.at[sl], sem
                    )
                )
            for c in copies:
                c.start()
            for c in copies:
                c.wait()

        pltpu.emit_pipeline(
            body,
            grid=(num // step_rows,),
            in_specs=[
                pl.BlockSpec((1, step_rows), lambda i: (0, i)),
            ],
            out_specs=[
                pl.BlockSpec((step_rows, depth), lambda i: (i, 0)),
            ],
            core_axis_name=("c", "s"),
            dimension_semantics=(pltpu.PARALLEL,),
        )(ids_hbm, out_hbm)

    return gather_kernel(table, ids2)


_PACK = 8


def _mlp_packed(xa, xe, w1a_big, w1e_big, b1_big, w2_big, b2_big, block_rows):
    rows = xa.shape[0]

    def body(xa_ref, xe_ref, w1a_ref, w1e_ref, b1_ref, w2_ref, b2_ref, o_ref):
        h = jnp.dot(xe_ref[...], w1e_ref[...], preferred_element_type=jnp.float32)
        h = h + jnp.dot(xa_ref[...], w1a_ref[...], preferred_element_type=jnp.float32)
        h = jnp.maximum(h + b1_ref[...], 0.0)
        o_ref[...] = (
            jnp.dot(h, w2_ref[...], preferred_element_type=jnp.float32) + b2_ref[...]
        )

    return pl.pallas_call(
        body,
        grid=(rows // block_rows,),
        in_specs=[
            pl.BlockSpec((block_rows, 128), lambda i: (i, 0)),
            pl.BlockSpec((block_rows, 128), lambda i: (i, 0)),
            pl.BlockSpec((128, 256), lambda i: (0, 0)),
            pl.BlockSpec((128, 256), lambda i: (0, 0)),
            pl.BlockSpec((1, 256), lambda i: (0, 0)),
            pl.BlockSpec((256, 256), lambda i: (0, 0)),
            pl.BlockSpec((1, 256), lambda i: (0, 0)),
        ],
        out_specs=pl.BlockSpec((block_rows, 256), lambda i: (i, 0)),
        out_shape=jax.ShapeDtypeStruct((rows, 256), jnp.float32),
    )(xa, xe, w1a_big, w1e_big, b1_big, w2_big, b2_big)


_N_CHUNKS = 4


def kernel(edge_attr, table, W1, b1, W2, b2):
    num = edge_attr.shape[0]
    depth = table.shape[1]
    hid = W1.shape[1]
    ids = edge_attr[:, 0].astype(jnp.int32)
    eye = jnp.eye(_PACK, dtype=W1.dtype)
    w1e_big = jnp.kron(eye, W1[:depth])
    w1n = jnp.concatenate([jnp.zeros((1, hid), W1.dtype), W1[depth:]], axis=0)
    w1a_big = jnp.kron(eye, w1n)
    w2_big = jnp.kron(eye, W2)
    b1_big = jnp.tile(b1, _PACK).reshape(1, _PACK * hid)
    b2_big = jnp.tile(b2, _PACK).reshape(1, _PACK * hid)
    nc = num // _N_CHUNKS
    outs = []
    for c in range(_N_CHUNKS):
        ea_c = jax.lax.slice(edge_attr, (c * nc, 0), ((c + 1) * nc, depth))
        ids_c = jax.lax.slice(ids, (c * nc,), ((c + 1) * nc,))
        emb_c = _sc_gather(table, ids_c)
        xa = ea_c.reshape(nc // _PACK, _PACK * depth)
        xe = emb_c.reshape(nc // _PACK, _PACK * depth)
        outs.append(
            _mlp_packed(xa, xe, w1a_big, w1e_big, b1_big, w2_big, b2_big,
                        block_rows=2000)
        )
    out = jnp.concatenate(outs, axis=0)
    return out.reshape(num, hid)


# speedup vs baseline: 2.0897x; 2.0897x over previous
import functools

import jax
import jax.numpy as jnp
from jax.experimental import pallas as pl
from jax.experimental.pallas import tpu as pltpu
from jax.experimental.pallas import tpu_sc as plsc

_GATHER_WINDOW = 128
_GATHER_K = 8


def _sc_gather_2d(table, ids2):
    num = ids2.shape[1]
    depth = table.shape[1]
    step_rows = _GATHER_WINDOW * _GATHER_K
    mesh = plsc.VectorSubcoreMesh(core_axis_name="c", subcore_axis_name="s")

    @functools.partial(
        pl.kernel,
        out_type=jax.ShapeDtypeStruct((num, depth), table.dtype),
        mesh=mesh,
        scratch_types=[pltpu.SemaphoreType.DMA],
        compiler_params=pltpu.CompilerParams(use_tc_tiling_on_sc=False),
    )
    def gather_kernel(table_hbm, ids_hbm, out_hbm, sem):
        def body(i_vmem, o_vmem):
            copies = []
            for j in range(_GATHER_K):
                sl = pl.ds(j * _GATHER_WINDOW, _GATHER_WINDOW)
                copies.append(
                    pltpu.make_async_copy(
                        table_hbm.at[i_vmem.at[0, sl]], o_vmem.at[sl], sem
                    )
                )
            for c in copies:
                c.start()
            for c in copies:
                c.wait()

        pltpu.emit_pipeline(
            body,
            grid=(num // step_rows,),
            in_specs=[
                pl.BlockSpec((1, step_rows), lambda i: (0, i)),
            ],
            out_specs=[
                pl.BlockSpec((step_rows, depth), lambda i: (i, 0)),
            ],
            core_axis_name=("c", "s"),
            dimension_semantics=(pltpu.PARALLEL,),
        )(ids_hbm, out_hbm)

    return gather_kernel(table, ids2)


def _sc_gather(table, ids):
    return _sc_gather_2d(table, ids.reshape(1, ids.shape[0]))


def _mlp_t(ea_t, emb, w1eT, w1nT, b1c, w2T, b2c, block_cols):
    feat, num = ea_t.shape
    depth = emb.shape[1]
    hid = w2T.shape[0]

    def body(ea_ref, emb_ref, w1e_ref, w1n_ref, b1_ref, w2_ref, b2_ref, o_ref):
        emb_t = jnp.transpose(emb_ref[...])
        h = jnp.dot(w1e_ref[...], emb_t, preferred_element_type=jnp.float32)
        h = h + jnp.dot(w1n_ref[...], ea_ref[...],
                        preferred_element_type=jnp.float32)
        h = jnp.maximum(h + b1_ref[...], 0.0)
        o_ref[...] = (
            jnp.dot(w2_ref[...], h, preferred_element_type=jnp.float32)
            + b2_ref[...]
        )

    return pl.pallas_call(
        body,
        grid=(num // block_cols,),
        in_specs=[
            pl.BlockSpec((feat, block_cols), lambda i: (0, i)),
            pl.BlockSpec((block_cols, depth), lambda i: (i, 0)),
            pl.BlockSpec((hid, depth), lambda i: (0, 0)),
            pl.BlockSpec((hid, feat), lambda i: (0, 0)),
            pl.BlockSpec((hid, 1), lambda i: (0, 0)),
            pl.BlockSpec((hid, hid), lambda i: (0, 0)),
            pl.BlockSpec((hid, 1), lambda i: (0, 0)),
        ],
        out_specs=pl.BlockSpec((hid, block_cols), lambda i: (0, i)),
        out_shape=jax.ShapeDtypeStruct((hid, num), jnp.float32),
    )(ea_t, emb, w1eT, w1nT, b1c, w2T, b2c)


def kernel(edge_attr, table, W1, b1, W2, b2):
    num = edge_attr.shape[0]
    depth = table.shape[1]
    hid = W1.shape[1]
    ea_t = edge_attr.T
    ids2 = ea_t[0:1, :].astype(jnp.int32)
    emb = _sc_gather_2d(table, ids2)
    w1eT = W1[:depth].T
    w1nT = jnp.concatenate(
        [jnp.zeros((1, hid), W1.dtype), W1[depth:]], axis=0
    ).T
    b1c = b1.reshape(hid, 1)
    b2c = b2.reshape(hid, 1)
    out_t = _mlp_t(ea_t, emb, w1eT, w1nT, b1c, w2T=W2.T, b2c=b2c,
                   block_cols=6400)
    return out_t.T
